# baseline (speedup 1.0000x reference)
import jax
import jax.numpy as jnp
from jax import lax
from jax.experimental import pallas as pl
from jax.experimental.pallas import tpu as pltpu
from jax.experimental.pallas import tpu_sc as plsc

NC = 2
NS = 16
LANES = 16
CHUNK = 128


def _pick_sub(w):
    for t in range(min(w, 128), 15, -1):
        if w % t == 0 and t % 16 == 0:
            return t
    raise ValueError(f"no valid sub-chunk for {w}")


def _gcn_graph_kernel(N, E, D, n_sel, tsub, interpret=False):
    HALF = D // 2
    NV = HALF // LANES
    EPS = E // NS
    NSUBT = EPS // CHUNK
    GROUP = max(g for g in range(1, 21) if NSUBT % g == 0)
    BIGE = GROUP * CHUNK
    NBIG = NSUBT // GROUP
    RPT = N // NS
    ZR = max(t for t in range(8, 65, 8) if RPT % t == 0)
    SELW = n_sel // NS
    NSELR = SELW // tsub
    f32 = jnp.float32
    assert E % NS == 0 and EPS % BIGE == 0
    assert N % NS == 0 and RPT % ZR == 0 and ZR % 8 == 0
    assert HALF % LANES == 0 and n_sel % (NS * tsub) == 0

    NSUB = NBIG * GROUP
    NB = 3

    def body(x0, src, dst, val, sel, x1, x2, x3, gsum,
             src_v, dst_v, val_v, rows2, zbuf, selv, rowsA,
             gix, dix, vch, sel_cur, gsem, ssem, acc_sh):
        c = lax.axis_index("c")
        s = lax.axis_index("s")
        xs = [x0, x1, x2, x3]

        def copy_idx(src1d, base, dst2d, b, width):
            for v_i in range(width // LANES):
                dst2d[b, pl.ds(LANES * v_i, LANES)] = (
                    src1d[pl.ds(base + LANES * v_i, LANES)])

        def zz(r, _):
            for v_i in range(NV):
                zbuf[r, pl.ds(LANES * v_i, LANES)] = jnp.zeros((LANES,), f32)
            return 0
        lax.fori_loop(0, ZR, zz, 0)

        def zero_slice():
            def zcopy(j, _):
                pltpu.sync_copy(zbuf, acc_sh.at[pl.ds(s * RPT + j * ZR, ZR)])
                return 0
            lax.fori_loop(0, RPT // ZR, zcopy, 0)

        def graph_prog(cc):
            zero_slice()
            plsc.subcore_barrier()

            for l in range(3):
                xin, xout = xs[l], xs[l + 1]

                def stage(t):
                    @pl.when(lax.rem(t, GROUP) == 0)
                    def _():
                        ebase = s * EPS + (t // GROUP) * BIGE
                        pltpu.sync_copy(src.at[pl.ds(ebase, BIGE)], src_v)
                        pltpu.sync_copy(dst.at[pl.ds(ebase, BIGE)], dst_v)
                        pltpu.sync_copy(val.at[pl.ds(ebase, BIGE)], val_v)

                def start_gather(t, b):
                    off = lax.rem(t, GROUP) * CHUNK
                    copy_idx(src_v, off, gix, b, CHUNK)
                    copy_idx(dst_v, off, dix, b, CHUNK)
                    for v_i in range(CHUNK // LANES):
                        vch[b, pl.ds(LANES * v_i, LANES)] = (
                            val_v[pl.ds(off + LANES * v_i, LANES)])
                    pltpu.async_copy(xin.at[cc].at[gix.at[b]], rows2.at[b],
                                     gsem.at[b])

                stage(0)
                start_gather(0, 0)
                stage(1)
                start_gather(1, 1)

                def sub(t, _):
                    b = lax.rem(t, NB)
                    pltpu.make_async_copy(xin.at[cc].at[gix.at[b]],
                                          rows2.at[b], gsem.at[b]).wait()

                    def sc_body(kk, _):
                        vv = vch[b, pl.ds(kk * LANES, LANES)]
                        for e in range(LANES):
                            v = vv[e]
                            row = kk * LANES + e
                            for v_i in range(NV):
                                sl = pl.ds(LANES * v_i, LANES)
                                rows2[b, row, sl] = rows2[b, row, sl] * v
                        return 0
                    lax.fori_loop(0, CHUNK // LANES, sc_body, 0)

                    @pl.when(t + 2 < NSUB)
                    def _():
                        stage(t + 2)
                        b2 = lax.rem(t + 2, NB)

                        @pl.when(t >= 1)
                        def _():
                            pltpu.make_async_copy(
                                rows2.at[b2], acc_sh.at[dix.at[b2]],
                                ssem.at[b2]).wait()
                        start_gather(t + 2, b2)

                    pltpu.async_copy(rows2.at[b], acc_sh.at[dix.at[b]],
                                     ssem.at[b], add=True)
                    return 0
                lax.fori_loop(0, NSUB, sub, 0)
                for bb in range(NB):
                    pltpu.make_async_copy(rows2.at[bb], acc_sh.at[dix.at[bb]],
                                          ssem.at[bb]).wait()

                plsc.subcore_barrier()
                pltpu.sync_copy(acc_sh.at[pl.ds(s * RPT, RPT)],
                                xout.at[cc, pl.ds(s * RPT, RPT)])
                if l < 2:
                    zero_slice()
                plsc.subcore_barrier()

            pltpu.sync_copy(sel.at[pl.ds(s * SELW, SELW)], selv)
            for r in range(NSELR):
                copy_idx(selv, r * tsub, sel_cur, 0, tsub)
                pltpu.sync_copy(x0.at[cc].at[sel_cur.at[0]],
                                rowsA.at[pl.ds(r * tsub, tsub)])
            for xr in (x1, x2, x3):
                for r in range(NSELR):
                    copy_idx(selv, r * tsub, sel_cur, 0, tsub)
                    pltpu.sync_copy(xr.at[cc].at[sel_cur.at[0]],
                                    rows2.at[0, pl.ds(0, tsub)])

                    def addb(k, _):
                        for v_i in range(NV):
                            sl = pl.ds(LANES * v_i, LANES)
                            rowsA[r * tsub + k, sl] = (
                                rowsA[r * tsub + k, sl] + rows2[0, k, sl])
                        return 0
                    lax.fori_loop(0, tsub, addb, 0)
            pltpu.sync_copy(rowsA, gsum.at[cc, pl.ds(s * SELW, SELW)])

        for cc in range(NC):
            @pl.when(c == cc)
            def _(cc=cc):
                graph_prog(cc)

    mesh = plsc.VectorSubcoreMesh(core_axis_name="c", subcore_axis_name="s",
                                  num_cores=NC, num_subcores=NS)
    sds = jax.ShapeDtypeStruct
    return pl.kernel(
        body,
        out_type=(sds((NC, N, HALF), f32), sds((NC, N, HALF), f32),
                  sds((NC, N, HALF), f32), sds((NC, n_sel, HALF), f32)),
        mesh=mesh,
        compiler_params=pltpu.CompilerParams(use_tc_tiling_on_sc=False),
        scratch_types=[
            pltpu.VMEM((BIGE,), jnp.int32),
            pltpu.VMEM((BIGE,), jnp.int32),
            pltpu.VMEM((BIGE,), f32),
            pltpu.VMEM((3, CHUNK, HALF), f32),
            pltpu.VMEM((ZR, HALF), f32),
            pltpu.VMEM((SELW,), jnp.int32),
            pltpu.VMEM((SELW, HALF), f32),
            pltpu.VMEM((3, CHUNK), jnp.int32),
            pltpu.VMEM((3, CHUNK), jnp.int32),
            pltpu.VMEM((3, CHUNK), f32),
            pltpu.VMEM((1, tsub), jnp.int32),
            pltpu.SemaphoreType.DMA((3,)),
            pltpu.SemaphoreType.DMA((3,)),
            pltpu.VMEM_SHARED((N, HALF), f32),
        ],
        interpret=interpret,
    )


def _l2h(lo, hi):
    n2 = (jnp.sum(lo * lo, axis=1, keepdims=True)
          + jnp.sum(hi * hi, axis=1, keepdims=True))
    inv = 1.0 / jnp.maximum(jnp.sqrt(n2), 1e-12)
    return lo * inv, hi * inv


def _tc_stage_a(B, NU, N, D, interpret=False):
    HALF = D // 2
    assert NU % 25 == 0
    BLK = NU // 25
    NBLK = NU // BLK
    f32 = jnp.float32

    def body(gsA, gs1, gs2, x0u, x1u, x2u, x3u, x0i, x1i, x2i, x3i,
             ue2n_ref, ie2n_ref, u1n_ref, i1n_ref, pu_ref, pi_ref, sup_ref):
        i = pl.program_id(0)
        q = f32(0.25)

        xu = (x0u[...] + x1u[...] + x2u[...] + x3u[...]) * q
        rl, rh = _l2h(xu[0], xu[1])
        ue2n_ref[0] = rl
        ue2n_ref[1] = rh
        xi = (x0i[...] + x1i[...] + x2i[...] + x3i[...]) * q
        sl, sh = _l2h(xi[0], xi[1])
        ie2n_ref[0] = sl
        ie2n_ref[1] = sh

        @pl.when(i == 0)
        def _():
            g1 = gs1[...]
            g2 = gs2[...]
            u1l, u1h = _l2h(g1[0, :B] * q, g1[1, :B] * q)
            i1l, i1h = _l2h(g1[0, B:] * q, g1[1, B:] * q)
            u2l, u2h = _l2h(g2[0, :B] * q, g2[1, :B] * q)
            i2l, i2h = _l2h(g2[0, B:] * q, g2[1, B:] * q)
            u1n_ref[0] = u1l
            u1n_ref[1] = u1h
            i1n_ref[0] = i1l
            i1n_ref[1] = i1h
            pu_ref[...] = jnp.sum(u1l * u2l + u1h * u2h, axis=1)[None, :]
            pi_ref[...] = jnp.sum(i1l * i2l + i1h * i2h, axis=1)[None, :]
            gA = gsA[...]
            ul, uh = gA[0, :B] * q, gA[1, :B] * q
            il, ih = gA[0, B:2 * B] * q, gA[1, B:2 * B] * q
            nl, nh = gA[0, 2 * B:] * q, gA[1, 2 * B:] * q
            sup = (jnp.sum(ul * il + uh * ih, axis=1)
                   - jnp.sum(ul * nl + uh * nh, axis=1))
            sup_ref[...] = sup[None, :]

    sds = jax.ShapeDtypeStruct
    full3 = lambda shape: pl.BlockSpec(shape, lambda i: (0, 0, 0))
    return pl.pallas_call(
        body,
        grid=(NBLK,),
        in_specs=[
            full3((NC, 3 * B, HALF)),
            full3((NC, 2 * B, HALF)),
            full3((NC, 2 * B, HALF)),
            *([pl.BlockSpec((NC, BLK, HALF), lambda i: (0, i, 0))] * 4),
            *([pl.BlockSpec((NC, BLK, HALF), lambda i: (0, i + NBLK, 0))] * 4),
        ],
        out_specs=[
            pl.BlockSpec((NC, BLK, HALF), lambda i: (0, i, 0)),
            pl.BlockSpec((NC, BLK, HALF), lambda i: (0, i, 0)),
            full3((NC, B, HALF)),
            full3((NC, B, HALF)),
            pl.BlockSpec((1, B), lambda i: (0, 0)),
            pl.BlockSpec((1, B), lambda i: (0, 0)),
            pl.BlockSpec((1, B), lambda i: (0, 0)),
        ],
        out_shape=[
            sds((NC, NU, HALF), f32), sds((NC, NU, HALF), f32),
            sds((NC, B, HALF), f32), sds((NC, B, HALF), f32),
            sds((1, B), f32), sds((1, B), f32), sds((1, B), f32),
        ],
        interpret=interpret,
    )


def _tc_matmul(B, NU, D, interpret=False):
    HALF = D // 2
    RB = 64
    assert B % RB == 0
    f32 = jnp.float32
    dn = (((1,), (1,)), ((), ()))

    def body(l1n, e2n, pos, out_ref):
        i = pl.program_id(0)
        l = l1n[...]
        r = e2n[...]
        acc = (lax.dot_general(l[0], r[0], dn, preferred_element_type=f32)
               + lax.dot_general(l[1], r[1], dn, preferred_element_type=f32))
        p = pos[i, :]
        out_ref[...] = acc - p[:, None]

    sds = jax.ShapeDtypeStruct
    return pl.pallas_call(
        body,
        grid=(B // RB,),
        in_specs=[
            pl.BlockSpec((NC, RB, HALF), lambda i: (0, i, 0)),
            pl.BlockSpec((NC, NU, HALF), lambda i: (0, 0, 0)),
            pl.BlockSpec((B // RB, RB), lambda i: (0, 0)),
        ],
        out_specs=pl.BlockSpec((RB, NU), lambda i: (i, 0)),
        out_shape=sds((B, NU), f32),
        interpret=interpret,
    )


def _run(user_table, item_table, adj_indices, adj_values, sg1_indices,
         sg1_values, sg2_indices, sg2_values, users, items, neg_items,
         interpret=False):
    NU, D = user_table.shape
    NI = item_table.shape[0]
    N = NU + NI
    E = adj_values.shape[0]
    B = users.shape[0]
    HALF = D // 2

    RPT = ((N // NS + 63) // 64) * 64
    N_PAD = RPT * NS
    ego = jnp.concatenate([user_table, item_table], axis=0)
    if N_PAD != N:
        ego = jnp.pad(ego, ((0, N_PAD - N), (0, 0)))
    x0 = jnp.stack([ego[:, :HALF], ego[:, HALF:]], axis=0)
    items_n = items + NU
    neg_n = neg_items + NU
    sel_adj = jnp.concatenate([users, items_n, neg_n])
    sel_ssl = jnp.concatenate([users, items_n])

    ts_a = _pick_sub(3 * B // NS)
    ts_s = _pick_sub(2 * B // NS)

    E_PAD = ((E + NS * CHUNK - 1) // (NS * CHUNK)) * (NS * CHUNK)

    def pe(a):
        return jnp.pad(a, (0, E_PAD - E)) if E_PAD != E else a

    gk_a = _gcn_graph_kernel(N_PAD, E_PAD, D, 3 * B, ts_a, interpret=interpret)
    gk_s = _gcn_graph_kernel(N_PAD, E_PAD, D, 2 * B, ts_s, interpret=interpret)

    _, _, _, gsA = gk_a(x0, pe(adj_indices[1]), pe(adj_indices[0]),
                        pe(adj_values), sel_adj)
    _, _, _, gs1 = gk_s(x0, pe(sg1_indices[1]), pe(sg1_indices[0]),
                        pe(sg1_values), sel_ssl)
    x1, x2, x3, gs2 = gk_s(x0, pe(sg2_indices[1]), pe(sg2_indices[0]),
                           pe(sg2_values), sel_ssl)

    ta = _tc_stage_a(B, NU, N, D, interpret=interpret)
    ue2n, ie2n, u1n, i1n, pu, pi, sup2 = ta(
        gsA, gs1, gs2, x0, x1, x2, x3, x0, x1, x2, x3)
    tm = _tc_matmul(B, NU, D, interpret=interpret)
    sslu = tm(u1n, ue2n, pu.reshape(-1, 64))
    ssli = tm(i1n, ie2n, pi.reshape(-1, 64))
    return (sup2.reshape(B), sslu, ssli)


def kernel(user_table, item_table, adj_indices, adj_values, sg1_indices,
           sg1_values, sg2_indices, sg2_values, users, items, neg_items):
    return _run(user_table, item_table, adj_indices, adj_values, sg1_indices,
                sg1_values, sg2_indices, sg2_values, users, items, neg_items)


# speedup vs baseline: 1.1816x; 1.1816x over previous
import jax
import jax.numpy as jnp
from jax import lax
from jax.experimental import pallas as pl
from jax.experimental.pallas import tpu as pltpu
from jax.experimental.pallas import tpu_sc as plsc

NC = 2
NS = 16
LANES = 16
CHUNK = 128


def _pick_sub(w):
    for t in range(min(w, 128), 15, -1):
        if w % t == 0 and t % 16 == 0:
            return t
    raise ValueError(f"no valid sub-chunk for {w}")


def _gcn_graph_kernel(N, E, D, n_sel, tsub, interpret=False):
    HALF = D // 2
    NV = HALF // LANES
    EPS = E // NS
    NSUBT = EPS // CHUNK
    GROUP = max(g for g in range(1, 21) if NSUBT % g == 0)
    BIGE = GROUP * CHUNK
    NBIG = NSUBT // GROUP
    RPT = N // NS
    ZR = max(t for t in range(8, 65, 8) if RPT % t == 0)
    SELW = n_sel // NS
    NSELR = SELW // tsub
    f32 = jnp.float32
    assert E % NS == 0 and EPS % BIGE == 0
    assert N % NS == 0 and RPT % ZR == 0 and ZR % 8 == 0
    assert HALF % LANES == 0 and n_sel % (NS * tsub) == 0

    NSUB = NBIG * GROUP
    NB = 3

    def body(x0, src, dst, val, sel, x1, x2, x3, gsum,
             src_v, dst_v, val_v, rows2, zbuf, selv, rowsA,
             gix, dix, sel_cur, gsem, ssem, acc_sh):
        c = lax.axis_index("c")
        s = lax.axis_index("s")
        xs = [x0, x1, x2, x3]

        def copy_idx(src1d, base, dst2d, b, width):
            for v_i in range(width // LANES):
                dst2d[b, pl.ds(LANES * v_i, LANES)] = (
                    src1d[pl.ds(base + LANES * v_i, LANES)])

        def zz(r, _):
            for v_i in range(NV):
                zbuf[r, pl.ds(LANES * v_i, LANES)] = jnp.zeros((LANES,), f32)
            return 0
        lax.fori_loop(0, ZR, zz, 0)

        def zero_slice():
            def zcopy(j, _):
                pltpu.sync_copy(zbuf, acc_sh.at[pl.ds(s * RPT + j * ZR, ZR)])
                return 0
            lax.fori_loop(0, RPT // ZR, zcopy, 0)

        def graph_prog(cc):
            zero_slice()
            plsc.subcore_barrier()

            for l in range(3):
                xin, xout = xs[l], xs[l + 1]

                def stage(t):
                    @pl.when(lax.rem(t, GROUP) == 0)
                    def _():
                        ebase = s * EPS + (t // GROUP) * BIGE
                        pltpu.sync_copy(src.at[pl.ds(ebase, BIGE)], src_v)
                        pltpu.sync_copy(dst.at[pl.ds(ebase, BIGE)], dst_v)
                        pltpu.sync_copy(val.at[pl.ds(ebase, BIGE)], val_v)

                def start_gather(t, b):
                    copy_idx(src_v, lax.rem(t, GROUP) * CHUNK, gix, b, CHUNK)
                    pltpu.async_copy(xin.at[cc].at[gix.at[b]], rows2.at[b],
                                     gsem.at[b])

                stage(0)
                start_gather(0, 0)

                def sub(t, _):
                    b = lax.rem(t, 2)
                    nb = 1 - b
                    off = lax.rem(t, GROUP) * CHUNK
                    pltpu.make_async_copy(xin.at[cc].at[gix.at[b]],
                                          rows2.at[b], gsem.at[b]).wait()

                    def sc_body(kk, _):
                        vv = val_v[pl.ds(off + kk * LANES, LANES)]
                        for e in range(LANES):
                            v = vv[e]
                            row = kk * LANES + e
                            for v_i in range(NV):
                                sl = pl.ds(LANES * v_i, LANES)
                                rows2[b, row, sl] = rows2[b, row, sl] * v
                        return 0
                    lax.fori_loop(0, CHUNK // LANES, sc_body, 0)

                    copy_idx(dst_v, off, dix, b, CHUNK)

                    @pl.when(t + 1 < NSUB)
                    def _():
                        stage(t + 1)

                        @pl.when(t >= 1)
                        def _():
                            pltpu.make_async_copy(
                                rows2.at[nb], acc_sh.at[dix.at[nb]],
                                ssem.at[nb]).wait()
                        start_gather(t + 1, nb)

                    pltpu.async_copy(rows2.at[b], acc_sh.at[dix.at[b]],
                                     ssem.at[b], add=True)
                    return 0
                lax.fori_loop(0, NSUB, sub, 0)
                for bb in range(2):
                    pltpu.make_async_copy(rows2.at[bb], acc_sh.at[dix.at[bb]],
                                          ssem.at[bb]).wait()

                plsc.subcore_barrier()
                pltpu.sync_copy(acc_sh.at[pl.ds(s * RPT, RPT)],
                                xout.at[cc, pl.ds(s * RPT, RPT)])
                if l < 2:
                    zero_slice()
                plsc.subcore_barrier()

            pltpu.sync_copy(sel.at[pl.ds(s * SELW, SELW)], selv)
            for r in range(NSELR):
                copy_idx(selv, r * tsub, sel_cur, 0, tsub)
                pltpu.sync_copy(x0.at[cc].at[sel_cur.at[0]],
                                rowsA.at[pl.ds(r * tsub, tsub)])
            for xr in (x1, x2, x3):
                for r in range(NSELR):
                    copy_idx(selv, r * tsub, sel_cur, 0, tsub)
                    pltpu.sync_copy(xr.at[cc].at[sel_cur.at[0]],
                                    rows2.at[0, pl.ds(0, tsub)])

                    def addb(k, _):
                        for v_i in range(NV):
                            sl = pl.ds(LANES * v_i, LANES)
                            rowsA[r * tsub + k, sl] = (
                                rowsA[r * tsub + k, sl] + rows2[0, k, sl])
                        return 0
                    lax.fori_loop(0, tsub, addb, 0)
            pltpu.sync_copy(rowsA, gsum.at[cc, pl.ds(s * SELW, SELW)])

        for cc in range(NC):
            @pl.when(c == cc)
            def _(cc=cc):
                graph_prog(cc)

    mesh = plsc.VectorSubcoreMesh(core_axis_name="c", subcore_axis_name="s",
                                  num_cores=NC, num_subcores=NS)
    sds = jax.ShapeDtypeStruct
    return pl.kernel(
        body,
        out_type=(sds((NC, N, HALF), f32), sds((NC, N, HALF), f32),
                  sds((NC, N, HALF), f32), sds((NC, n_sel, HALF), f32)),
        mesh=mesh,
        compiler_params=pltpu.CompilerParams(use_tc_tiling_on_sc=False),
        scratch_types=[
            pltpu.VMEM((BIGE,), jnp.int32),
            pltpu.VMEM((BIGE,), jnp.int32),
            pltpu.VMEM((BIGE,), f32),
            pltpu.VMEM((2, CHUNK, HALF), f32),
            pltpu.VMEM((ZR, HALF), f32),
            pltpu.VMEM((SELW,), jnp.int32),
            pltpu.VMEM((SELW, HALF), f32),
            pltpu.VMEM((2, CHUNK), jnp.int32),
            pltpu.VMEM((2, CHUNK), jnp.int32),
            pltpu.VMEM((1, tsub), jnp.int32),
            pltpu.SemaphoreType.DMA((2,)),
            pltpu.SemaphoreType.DMA((2,)),
            pltpu.VMEM_SHARED((N, HALF), f32),
        ],
        interpret=interpret,
    )


def _l2h(lo, hi):
    n2 = (jnp.sum(lo * lo, axis=1, keepdims=True)
          + jnp.sum(hi * hi, axis=1, keepdims=True))
    inv = 1.0 / jnp.maximum(jnp.sqrt(n2), 1e-12)
    return lo * inv, hi * inv


def _tc_stage_a(B, NU, N, D, interpret=False):
    HALF = D // 2
    assert NU % 25 == 0
    BLK = NU // 25
    NBLK = NU // BLK
    f32 = jnp.float32

    def body(gsA, gs1, gs2, x0u, x1u, x2u, x3u, x0i, x1i, x2i, x3i,
             ue2n_ref, ie2n_ref, u1n_ref, i1n_ref, pu_ref, pi_ref, sup_ref):
        i = pl.program_id(0)
        q = f32(0.25)

        xu = (x0u[...] + x1u[...] + x2u[...] + x3u[...]) * q
        rl, rh = _l2h(xu[0], xu[1])
        ue2n_ref[0] = rl
        ue2n_ref[1] = rh
        xi = (x0i[...] + x1i[...] + x2i[...] + x3i[...]) * q
        sl, sh = _l2h(xi[0], xi[1])
        ie2n_ref[0] = sl
        ie2n_ref[1] = sh

        @pl.when(i == 0)
        def _():
            g1 = gs1[...]
            g2 = gs2[...]
            u1l, u1h = _l2h(g1[0, :B] * q, g1[1, :B] * q)
            i1l, i1h = _l2h(g1[0, B:] * q, g1[1, B:] * q)
            u2l, u2h = _l2h(g2[0, :B] * q, g2[1, :B] * q)
            i2l, i2h = _l2h(g2[0, B:] * q, g2[1, B:] * q)
            u1n_ref[0] = u1l
            u1n_ref[1] = u1h
            i1n_ref[0] = i1l
            i1n_ref[1] = i1h
            pu_ref[...] = jnp.sum(u1l * u2l + u1h * u2h, axis=1)[None, :]
            pi_ref[...] = jnp.sum(i1l * i2l + i1h * i2h, axis=1)[None, :]
            gA = gsA[...]
            ul, uh = gA[0, :B] * q, gA[1, :B] * q
            il, ih = gA[0, B:2 * B] * q, gA[1, B:2 * B] * q
            nl, nh = gA[0, 2 * B:] * q, gA[1, 2 * B:] * q
            sup = (jnp.sum(ul * il + uh * ih, axis=1)
                   - jnp.sum(ul * nl + uh * nh, axis=1))
            sup_ref[...] = sup[None, :]

    sds = jax.ShapeDtypeStruct
    full3 = lambda shape: pl.BlockSpec(shape, lambda i: (0, 0, 0))
    return pl.pallas_call(
        body,
        grid=(NBLK,),
        in_specs=[
            full3((NC, 3 * B, HALF)),
            full3((NC, 2 * B, HALF)),
            full3((NC, 2 * B, HALF)),
            *([pl.BlockSpec((NC, BLK, HALF), lambda i: (0, i, 0))] * 4),
            *([pl.BlockSpec((NC, BLK, HALF), lambda i: (0, i + NBLK, 0))] * 4),
        ],
        out_specs=[
            pl.BlockSpec((NC, BLK, HALF), lambda i: (0, i, 0)),
            pl.BlockSpec((NC, BLK, HALF), lambda i: (0, i, 0)),
            full3((NC, B, HALF)),
            full3((NC, B, HALF)),
            pl.BlockSpec((1, B), lambda i: (0, 0)),
            pl.BlockSpec((1, B), lambda i: (0, 0)),
            pl.BlockSpec((1, B), lambda i: (0, 0)),
        ],
        out_shape=[
            sds((NC, NU, HALF), f32), sds((NC, NU, HALF), f32),
            sds((NC, B, HALF), f32), sds((NC, B, HALF), f32),
            sds((1, B), f32), sds((1, B), f32), sds((1, B), f32),
        ],
        interpret=interpret,
    )


def _tc_matmul(B, NU, D, interpret=False):
    HALF = D // 2
    RB = 64
    assert B % RB == 0
    f32 = jnp.float32
    dn = (((1,), (1,)), ((), ()))

    def body(l1n, e2n, pos, out_ref):
        i = pl.program_id(0)
        l = l1n[...]
        r = e2n[...]
        acc = (lax.dot_general(l[0], r[0], dn, preferred_element_type=f32)
               + lax.dot_general(l[1], r[1], dn, preferred_element_type=f32))
        p = pos[i, :]
        out_ref[...] = acc - p[:, None]

    sds = jax.ShapeDtypeStruct
    return pl.pallas_call(
        body,
        grid=(B // RB,),
        in_specs=[
            pl.BlockSpec((NC, RB, HALF), lambda i: (0, i, 0)),
            pl.BlockSpec((NC, NU, HALF), lambda i: (0, 0, 0)),
            pl.BlockSpec((B // RB, RB), lambda i: (0, 0)),
        ],
        out_specs=pl.BlockSpec((RB, NU), lambda i: (i, 0)),
        out_shape=sds((B, NU), f32),
        interpret=interpret,
    )


def _run(user_table, item_table, adj_indices, adj_values, sg1_indices,
         sg1_values, sg2_indices, sg2_values, users, items, neg_items,
         interpret=False):
    NU, D = user_table.shape
    NI = item_table.shape[0]
    N = NU + NI
    E = adj_values.shape[0]
    B = users.shape[0]
    HALF = D // 2

    RPT = ((N // NS + 63) // 64) * 64
    N_PAD = RPT * NS
    ego = jnp.concatenate([user_table, item_table], axis=0)
    if N_PAD != N:
        ego = jnp.pad(ego, ((0, N_PAD - N), (0, 0)))
    x0 = jnp.stack([ego[:, :HALF], ego[:, HALF:]], axis=0)
    items_n = items + NU
    neg_n = neg_items + NU
    sel_adj = jnp.concatenate([users, items_n, neg_n])
    sel_ssl = jnp.concatenate([users, items_n])

    ts_a = _pick_sub(3 * B // NS)
    ts_s = _pick_sub(2 * B // NS)

    E_PAD = ((E + NS * CHUNK - 1) // (NS * CHUNK)) * (NS * CHUNK)

    def pe(a):
        return jnp.pad(a, (0, E_PAD - E)) if E_PAD != E else a

    gk_a = _gcn_graph_kernel(N_PAD, E_PAD, D, 3 * B, ts_a, interpret=interpret)
    gk_s = _gcn_graph_kernel(N_PAD, E_PAD, D, 2 * B, ts_s, interpret=interpret)

    _, _, _, gsA = gk_a(x0, pe(adj_indices[1]), pe(adj_indices[0]),
                        pe(adj_values), sel_adj)
    _, _, _, gs1 = gk_s(x0, pe(sg1_indices[1]), pe(sg1_indices[0]),
                        pe(sg1_values), sel_ssl)
    x1, x2, x3, gs2 = gk_s(x0, pe(sg2_indices[1]), pe(sg2_indices[0]),
                           pe(sg2_values), sel_ssl)

    ta = _tc_stage_a(B, NU, N, D, interpret=interpret)
    ue2n, ie2n, u1n, i1n, pu, pi, sup2 = ta(
        gsA, gs1, gs2, x0, x1, x2, x3, x0, x1, x2, x3)
    tm = _tc_matmul(B, NU, D, interpret=interpret)
    sslu = tm(u1n, ue2n, pu.reshape(-1, 64))
    ssli = tm(i1n, ie2n, pi.reshape(-1, 64))
    return (sup2.reshape(B), sslu, ssli)


def kernel(user_table, item_table, adj_indices, adj_values, sg1_indices,
           sg1_values, sg2_indices, sg2_values, users, items, neg_items):
    return _run(user_table, item_table, adj_indices, adj_values, sg1_indices,
                sg1_values, sg2_indices, sg2_values, users, items, neg_items)


# speedup vs baseline: 1.2024x; 1.0176x over previous
import jax
import jax.numpy as jnp
from jax import lax
from jax.experimental import pallas as pl
from jax.experimental.pallas import tpu as pltpu
from jax.experimental.pallas import tpu_sc as plsc

NC = 2
NS = 16
LANES = 16
CHUNK = 128


def _pick_sub(w):
    for t in range(min(w, 128), 15, -1):
        if w % t == 0 and t % 16 == 0:
            return t
    raise ValueError(f"no valid sub-chunk for {w}")


def _gcn_graph_kernel(N, E, D, n_sel, tsub, interpret=False):
    HALF = D // 2
    NV = HALF // LANES
    EPS = E // NS
    NSUBT = EPS // CHUNK
    GROUP = max(g for g in range(1, 26) if NSUBT % g == 0)
    BIGE = GROUP * CHUNK
    NBIG = NSUBT // GROUP
    RPT = N // NS
    ZR = max(t for t in range(8, 129, 8) if RPT % t == 0)
    SELW = n_sel // NS
    NSELR = SELW // tsub
    f32 = jnp.float32
    assert E % NS == 0 and EPS % BIGE == 0
    assert N % NS == 0 and RPT % ZR == 0 and ZR % 8 == 0
    assert HALF % LANES == 0 and n_sel % (NS * tsub) == 0

    NSUB = NBIG * GROUP
    NB = 3

    def body(x0, src, dst, val, sel, x1, x2, x3, gsum,
             src_v, dst_v, val_v, rows2, zbuf, selv, rowsA,
             gix, dix, sel_cur, gsem, ssem, acc_sh):
        c = lax.axis_index("c")
        s = lax.axis_index("s")
        xs = [x0, x1, x2, x3]

        def copy_idx(src1d, base, dst2d, b, width):
            for v_i in range(width // LANES):
                dst2d[b, pl.ds(LANES * v_i, LANES)] = (
                    src1d[pl.ds(base + LANES * v_i, LANES)])

        def zz(r, _):
            for v_i in range(NV):
                zbuf[r, pl.ds(LANES * v_i, LANES)] = jnp.zeros((LANES,), f32)
            return 0
        lax.fori_loop(0, ZR, zz, 0)

        def zero_slice():
            def zcopy(j, _):
                pltpu.sync_copy(zbuf, acc_sh.at[pl.ds(s * RPT + j * ZR, ZR)])
                return 0
            lax.fori_loop(0, RPT // ZR, zcopy, 0)

        def graph_prog(cc):
            zero_slice()
            plsc.subcore_barrier()

            for l in range(3):
                xin, xout = xs[l], xs[l + 1]

                def stage(t):
                    @pl.when(lax.rem(t, GROUP) == 0)
                    def _():
                        ebase = s * EPS + (t // GROUP) * BIGE
                        pltpu.sync_copy(src.at[pl.ds(ebase, BIGE)], src_v)
                        pltpu.sync_copy(dst.at[pl.ds(ebase, BIGE)], dst_v)
                        pltpu.sync_copy(val.at[pl.ds(ebase, BIGE)], val_v)

                def start_gather(t, b):
                    copy_idx(src_v, lax.rem(t, GROUP) * CHUNK, gix, b, CHUNK)
                    pltpu.async_copy(xin.at[cc].at[gix.at[b]], rows2.at[b],
                                     gsem.at[b])

                stage(0)
                start_gather(0, 0)

                def sub(t, _):
                    b = lax.rem(t, 2)
                    nb = 1 - b
                    off = lax.rem(t, GROUP) * CHUNK
                    pltpu.make_async_copy(xin.at[cc].at[gix.at[b]],
                                          rows2.at[b], gsem.at[b]).wait()

                    def sc_body(kk, _):
                        vv = val_v[pl.ds(off + kk * LANES, LANES)]
                        for e in range(LANES):
                            v = vv[e]
                            row = kk * LANES + e
                            for v_i in range(NV):
                                sl = pl.ds(LANES * v_i, LANES)
                                rows2[b, row, sl] = rows2[b, row, sl] * v
                        return 0
                    lax.fori_loop(0, CHUNK // LANES, sc_body, 0)

                    copy_idx(dst_v, off, dix, b, CHUNK)

                    @pl.when(t + 1 < NSUB)
                    def _():
                        stage(t + 1)

                        @pl.when(t >= 1)
                        def _():
                            pltpu.make_async_copy(
                                rows2.at[nb], acc_sh.at[dix.at[nb]],
                                ssem.at[nb]).wait()
                        start_gather(t + 1, nb)

                    pltpu.async_copy(rows2.at[b], acc_sh.at[dix.at[b]],
                                     ssem.at[b], add=True)
                    return 0
                lax.fori_loop(0, NSUB, sub, 0)
                for bb in range(2):
                    pltpu.make_async_copy(rows2.at[bb], acc_sh.at[dix.at[bb]],
                                          ssem.at[bb]).wait()

                plsc.subcore_barrier()
                pltpu.sync_copy(acc_sh.at[pl.ds(s * RPT, RPT)],
                                xout.at[cc, pl.ds(s * RPT, RPT)])
                if l < 2:
                    zero_slice()
                plsc.subcore_barrier()

            pltpu.sync_copy(sel.at[pl.ds(s * SELW, SELW)], selv)
            for r in range(NSELR):
                copy_idx(selv, r * tsub, sel_cur, 0, tsub)
                pltpu.sync_copy(x0.at[cc].at[sel_cur.at[0]],
                                rowsA.at[pl.ds(r * tsub, tsub)])
            for xr in (x1, x2, x3):
                for r in range(NSELR):
                    copy_idx(selv, r * tsub, sel_cur, 0, tsub)
                    pltpu.sync_copy(xr.at[cc].at[sel_cur.at[0]],
                                    rows2.at[0, pl.ds(0, tsub)])

                    def addb(k, _):
                        for v_i in range(NV):
                            sl = pl.ds(LANES * v_i, LANES)
                            rowsA[r * tsub + k, sl] = (
                                rowsA[r * tsub + k, sl] + rows2[0, k, sl])
                        return 0
                    lax.fori_loop(0, tsub, addb, 0)
            pltpu.sync_copy(rowsA, gsum.at[cc, pl.ds(s * SELW, SELW)])

        for cc in range(NC):
            @pl.when(c == cc)
            def _(cc=cc):
                graph_prog(cc)

    mesh = plsc.VectorSubcoreMesh(core_axis_name="c", subcore_axis_name="s",
                                  num_cores=NC, num_subcores=NS)
    sds = jax.ShapeDtypeStruct
    return pl.kernel(
        body,
        out_type=(sds((NC, N, HALF), f32), sds((NC, N, HALF), f32),
                  sds((NC, N, HALF), f32), sds((NC, n_sel, HALF), f32)),
        mesh=mesh,
        compiler_params=pltpu.CompilerParams(use_tc_tiling_on_sc=False),
        scratch_types=[
            pltpu.VMEM((BIGE,), jnp.int32),
            pltpu.VMEM((BIGE,), jnp.int32),
            pltpu.VMEM((BIGE,), f32),
            pltpu.VMEM((2, CHUNK, HALF), f32),
            pltpu.VMEM((ZR, HALF), f32),
            pltpu.VMEM((SELW,), jnp.int32),
            pltpu.VMEM((SELW, HALF), f32),
            pltpu.VMEM((2, CHUNK), jnp.int32),
            pltpu.VMEM((2, CHUNK), jnp.int32),
            pltpu.VMEM((1, tsub), jnp.int32),
            pltpu.SemaphoreType.DMA((2,)),
            pltpu.SemaphoreType.DMA((2,)),
            pltpu.VMEM_SHARED((N, HALF), f32),
        ],
        interpret=interpret,
    )


def _l2h(lo, hi):
    n2 = (jnp.sum(lo * lo, axis=1, keepdims=True)
          + jnp.sum(hi * hi, axis=1, keepdims=True))
    inv = 1.0 / jnp.maximum(jnp.sqrt(n2), 1e-12)
    return lo * inv, hi * inv


def _tc_stage_a(B, NU, N, D, interpret=False):
    HALF = D // 2
    assert NU % 25 == 0
    BLK = NU // 25
    NBLK = NU // BLK
    f32 = jnp.float32

    def body(gsA, gs1, gs2, x0u, x1u, x2u, x3u, x0i, x1i, x2i, x3i,
             ue2n_ref, ie2n_ref, u1n_ref, i1n_ref, pu_ref, pi_ref, sup_ref):
        i = pl.program_id(0)
        q = f32(0.25)

        xu = (x0u[...] + x1u[...] + x2u[...] + x3u[...]) * q
        rl, rh = _l2h(xu[0], xu[1])
        ue2n_ref[0] = rl
        ue2n_ref[1] = rh
        xi = (x0i[...] + x1i[...] + x2i[...] + x3i[...]) * q
        sl, sh = _l2h(xi[0], xi[1])
        ie2n_ref[0] = sl
        ie2n_ref[1] = sh

        @pl.when(i == 0)
        def _():
            g1 = gs1[...]
            g2 = gs2[...]
            u1l, u1h = _l2h(g1[0, :B] * q, g1[1, :B] * q)
            i1l, i1h = _l2h(g1[0, B:] * q, g1[1, B:] * q)
            u2l, u2h = _l2h(g2[0, :B] * q, g2[1, :B] * q)
            i2l, i2h = _l2h(g2[0, B:] * q, g2[1, B:] * q)
            u1n_ref[0] = u1l
            u1n_ref[1] = u1h
            i1n_ref[0] = i1l
            i1n_ref[1] = i1h
            pu_ref[...] = jnp.sum(u1l * u2l + u1h * u2h, axis=1)[None, :]
            pi_ref[...] = jnp.sum(i1l * i2l + i1h * i2h, axis=1)[None, :]
            gA = gsA[...]
            ul, uh = gA[0, :B] * q, gA[1, :B] * q
            il, ih = gA[0, B:2 * B] * q, gA[1, B:2 * B] * q
            nl, nh = gA[0, 2 * B:] * q, gA[1, 2 * B:] * q
            sup = (jnp.sum(ul * il + uh * ih, axis=1)
                   - jnp.sum(ul * nl + uh * nh, axis=1))
            sup_ref[...] = sup[None, :]

    sds = jax.ShapeDtypeStruct
    full3 = lambda shape: pl.BlockSpec(shape, lambda i: (0, 0, 0))
    return pl.pallas_call(
        body,
        grid=(NBLK,),
        in_specs=[
            full3((NC, 3 * B, HALF)),
            full3((NC, 2 * B, HALF)),
            full3((NC, 2 * B, HALF)),
            *([pl.BlockSpec((NC, BLK, HALF), lambda i: (0, i, 0))] * 4),
            *([pl.BlockSpec((NC, BLK, HALF), lambda i: (0, i + NBLK, 0))] * 4),
        ],
        out_specs=[
            pl.BlockSpec((NC, BLK, HALF), lambda i: (0, i, 0)),
            pl.BlockSpec((NC, BLK, HALF), lambda i: (0, i, 0)),
            full3((NC, B, HALF)),
            full3((NC, B, HALF)),
            pl.BlockSpec((1, B), lambda i: (0, 0)),
            pl.BlockSpec((1, B), lambda i: (0, 0)),
            pl.BlockSpec((1, B), lambda i: (0, 0)),
        ],
        out_shape=[
            sds((NC, NU, HALF), f32), sds((NC, NU, HALF), f32),
            sds((NC, B, HALF), f32), sds((NC, B, HALF), f32),
            sds((1, B), f32), sds((1, B), f32), sds((1, B), f32),
        ],
        interpret=interpret,
    )


def _tc_matmul(B, NU, D, interpret=False):
    HALF = D // 2
    RB = 64
    assert B % RB == 0
    f32 = jnp.float32
    dn = (((1,), (1,)), ((), ()))

    def body(l1n, e2n, pos, out_ref):
        i = pl.program_id(0)
        l = l1n[...]
        r = e2n[...]
        acc = (lax.dot_general(l[0], r[0], dn, preferred_element_type=f32)
               + lax.dot_general(l[1], r[1], dn, preferred_element_type=f32))
        p = pos[i, :]
        out_ref[...] = acc - p[:, None]

    sds = jax.ShapeDtypeStruct
    return pl.pallas_call(
        body,
        grid=(B // RB,),
        in_specs=[
            pl.BlockSpec((NC, RB, HALF), lambda i: (0, i, 0)),
            pl.BlockSpec((NC, NU, HALF), lambda i: (0, 0, 0)),
            pl.BlockSpec((B // RB, RB), lambda i: (0, 0)),
        ],
        out_specs=pl.BlockSpec((RB, NU), lambda i: (i, 0)),
        out_shape=sds((B, NU), f32),
        interpret=interpret,
    )


def _run(user_table, item_table, adj_indices, adj_values, sg1_indices,
         sg1_values, sg2_indices, sg2_values, users, items, neg_items,
         interpret=False):
    NU, D = user_table.shape
    NI = item_table.shape[0]
    N = NU + NI
    E = adj_values.shape[0]
    B = users.shape[0]
    HALF = D // 2

    RPT = ((N // NS + 63) // 64) * 64
    N_PAD = RPT * NS
    ego = jnp.concatenate([user_table, item_table], axis=0)
    if N_PAD != N:
        ego = jnp.pad(ego, ((0, N_PAD - N), (0, 0)))
    x0 = jnp.stack([ego[:, :HALF], ego[:, HALF:]], axis=0)
    items_n = items + NU
    neg_n = neg_items + NU
    sel_adj = jnp.concatenate([users, items_n, neg_n])
    sel_ssl = jnp.concatenate([users, items_n])

    ts_a = _pick_sub(3 * B // NS)
    ts_s = _pick_sub(2 * B // NS)

    E_PAD = ((E + NS * CHUNK - 1) // (NS * CHUNK)) * (NS * CHUNK)

    def pe(a):
        return jnp.pad(a, (0, E_PAD - E)) if E_PAD != E else a

    gk_a = _gcn_graph_kernel(N_PAD, E_PAD, D, 3 * B, ts_a, interpret=interpret)
    gk_s = _gcn_graph_kernel(N_PAD, E_PAD, D, 2 * B, ts_s, interpret=interpret)

    _, _, _, gsA = gk_a(x0, pe(adj_indices[1]), pe(adj_indices[0]),
                        pe(adj_values), sel_adj)
    _, _, _, gs1 = gk_s(x0, pe(sg1_indices[1]), pe(sg1_indices[0]),
                        pe(sg1_values), sel_ssl)
    x1, x2, x3, gs2 = gk_s(x0, pe(sg2_indices[1]), pe(sg2_indices[0]),
                           pe(sg2_values), sel_ssl)

    ta = _tc_stage_a(B, NU, N, D, interpret=interpret)
    ue2n, ie2n, u1n, i1n, pu, pi, sup2 = ta(
        gsA, gs1, gs2, x0, x1, x2, x3, x0, x1, x2, x3)
    tm = _tc_matmul(B, NU, D, interpret=interpret)
    sslu = tm(u1n, ue2n, pu.reshape(-1, 64))
    ssli = tm(i1n, ie2n, pi.reshape(-1, 64))
    return (sup2.reshape(B), sslu, ssli)


def kernel(user_table, item_table, adj_indices, adj_values, sg1_indices,
           sg1_values, sg2_indices, sg2_values, users, items, neg_items):
    return _run(user_table, item_table, adj_indices, adj_values, sg1_indices,
                sg1_values, sg2_indices, sg2_values, users, items, neg_items)


# speedup vs baseline: 1.3632x; 1.1337x over previous
import jax
import jax.numpy as jnp
from jax import lax
from jax.experimental import pallas as pl
from jax.experimental.pallas import tpu as pltpu
from jax.experimental.pallas import tpu_sc as plsc

NC = 2
NS = 16
LANES = 16
CHUNK = 128


def _pick_sub(w):
    for t in range(min(w, 128), 15, -1):
        if w % t == 0 and t % 16 == 0:
            return t
    raise ValueError(f"no valid sub-chunk for {w}")


def _gcn_graph_kernel(N, E, D, n_sel, tsub, interpret=False):
    HALF = D // 2
    NV = HALF // LANES
    EPS = E // NS
    NSUBT = EPS // CHUNK
    GROUP = max(g for g in range(1, 26) if NSUBT % g == 0)
    BIGE = GROUP * CHUNK
    NBIG = NSUBT // GROUP
    RPT = N // NS
    ZR = max(t for t in range(8, 129, 8) if RPT % t == 0)
    SELW = n_sel // NS
    NSELR = SELW // tsub
    f32 = jnp.float32
    assert E % NS == 0 and EPS % BIGE == 0
    assert N % NS == 0 and RPT % ZR == 0 and ZR % 8 == 0
    assert HALF % LANES == 0 and n_sel % (NS * tsub) == 0

    NSUB = NBIG * GROUP
    NB = 3

    def body(x0, src, dst, val, sel, x1, x2, x3, gsum,
             src_v, dst_v, val_v, rows2, zbuf, selv, rowsA,
             gix, dix, vch, sel_cur, gsem, ssem, acc_sh):
        c = lax.axis_index("c")
        s = lax.axis_index("s")
        xs = [x0, x1, x2, x3]

        def copy_idx(src1d, base, dst2d, b, width):
            for v_i in range(width // LANES):
                dst2d[b, pl.ds(LANES * v_i, LANES)] = (
                    src1d[pl.ds(base + LANES * v_i, LANES)])

        def zz(r, _):
            for v_i in range(NV):
                zbuf[r, pl.ds(LANES * v_i, LANES)] = jnp.zeros((LANES,), f32)
            return 0
        lax.fori_loop(0, ZR, zz, 0)

        def zero_slice():
            def zcopy(j, _):
                pltpu.sync_copy(zbuf, acc_sh.at[pl.ds(s * RPT + j * ZR, ZR)])
                return 0
            lax.fori_loop(0, RPT // ZR, zcopy, 0)

        def graph_prog(cc):
            zero_slice()
            plsc.subcore_barrier()

            for l in range(3):
                xin, xout = xs[l], xs[l + 1]

                def stage(t):
                    @pl.when(lax.rem(t, GROUP) == 0)
                    def _():
                        ebase = s * EPS + (t // GROUP) * BIGE
                        pltpu.sync_copy(src.at[pl.ds(ebase, BIGE)], src_v)
                        pltpu.sync_copy(dst.at[pl.ds(ebase, BIGE)], dst_v)
                        pltpu.sync_copy(val.at[pl.ds(ebase, BIGE)], val_v)

                def start_gather(t, b):
                    copy_idx(src_v, lax.rem(t, GROUP) * CHUNK, gix, b, CHUNK)
                    pltpu.async_copy(xin.at[cc].at[gix.at[b]], rows2.at[b],
                                     gsem.at[b])

                stage(0)
                start_gather(0, 0)

                def sub(t, _):
                    b = lax.rem(t, 2)
                    nb = 1 - b
                    off = lax.rem(t, GROUP) * CHUNK
                    pltpu.make_async_copy(xin.at[cc].at[gix.at[b]],
                                          rows2.at[b], gsem.at[b]).wait()

                    for v_i in range(CHUNK // LANES):
                        vch[0, pl.ds(LANES * v_i, LANES)] = (
                            val_v[pl.ds(off + LANES * v_i, LANES)])
                    copy_idx(dst_v, off, dix, b, CHUNK)

                    @pl.when(t + 1 < NSUB)
                    def _():
                        stage(t + 1)

                        @pl.when(t >= 1)
                        def _():
                            pltpu.make_async_copy(
                                rows2.at[nb], acc_sh.at[dix.at[nb]],
                                ssem.at[nb]).wait()
                        start_gather(t + 1, nb)

                    def sc_body(kk, _):
                        vv = vch[0, pl.ds(kk * LANES, LANES)]
                        for e in range(LANES):
                            v = vv[e]
                            row = kk * LANES + e
                            for v_i in range(NV):
                                sl = pl.ds(LANES * v_i, LANES)
                                rows2[b, row, sl] = rows2[b, row, sl] * v
                        return 0
                    lax.fori_loop(0, CHUNK // LANES, sc_body, 0)

                    pltpu.async_copy(rows2.at[b], acc_sh.at[dix.at[b]],
                                     ssem.at[b], add=True)
                    return 0
                lax.fori_loop(0, NSUB, sub, 0)
                for bb in range(2):
                    pltpu.make_async_copy(rows2.at[bb], acc_sh.at[dix.at[bb]],
                                          ssem.at[bb]).wait()

                plsc.subcore_barrier()
                pltpu.sync_copy(acc_sh.at[pl.ds(s * RPT, RPT)],
                                xout.at[cc, pl.ds(s * RPT, RPT)])
                if l < 2:
                    zero_slice()
                plsc.subcore_barrier()

            pltpu.sync_copy(sel.at[pl.ds(s * SELW, SELW)], selv)
            for r in range(NSELR):
                copy_idx(selv, r * tsub, sel_cur, 0, tsub)
                pltpu.sync_copy(x0.at[cc].at[sel_cur.at[0]],
                                rowsA.at[pl.ds(r * tsub, tsub)])
            for xr in (x1, x2, x3):
                for r in range(NSELR):
                    copy_idx(selv, r * tsub, sel_cur, 0, tsub)
                    pltpu.sync_copy(xr.at[cc].at[sel_cur.at[0]],
                                    rows2.at[0, pl.ds(0, tsub)])

                    def addb(k, _):
                        for v_i in range(NV):
                            sl = pl.ds(LANES * v_i, LANES)
                            rowsA[r * tsub + k, sl] = (
                                rowsA[r * tsub + k, sl] + rows2[0, k, sl])
                        return 0
                    lax.fori_loop(0, tsub, addb, 0)
            pltpu.sync_copy(rowsA, gsum.at[cc, pl.ds(s * SELW, SELW)])

        for cc in range(NC):
            @pl.when(c == cc)
            def _(cc=cc):
                graph_prog(cc)

    mesh = plsc.VectorSubcoreMesh(core_axis_name="c", subcore_axis_name="s",
                                  num_cores=NC, num_subcores=NS)
    sds = jax.ShapeDtypeStruct
    return pl.kernel(
        body,
        out_type=(sds((NC, N, HALF), f32), sds((NC, N, HALF), f32),
                  sds((NC, N, HALF), f32), sds((NC, n_sel, HALF), f32)),
        mesh=mesh,
        compiler_params=pltpu.CompilerParams(use_tc_tiling_on_sc=False),
        scratch_types=[
            pltpu.VMEM((BIGE,), jnp.int32),
            pltpu.VMEM((BIGE,), jnp.int32),
            pltpu.VMEM((BIGE,), f32),
            pltpu.VMEM((2, CHUNK, HALF), f32),
            pltpu.VMEM((ZR, HALF), f32),
            pltpu.VMEM((SELW,), jnp.int32),
            pltpu.VMEM((SELW, HALF), f32),
            pltpu.VMEM((2, CHUNK), jnp.int32),
            pltpu.VMEM((2, CHUNK), jnp.int32),
            pltpu.VMEM((1, CHUNK), f32),
            pltpu.VMEM((1, tsub), jnp.int32),
            pltpu.SemaphoreType.DMA((2,)),
            pltpu.SemaphoreType.DMA((2,)),
            pltpu.VMEM_SHARED((N, HALF), f32),
        ],
        interpret=interpret,
    )


def _l2h(lo, hi):
    n2 = (jnp.sum(lo * lo, axis=1, keepdims=True)
          + jnp.sum(hi * hi, axis=1, keepdims=True))
    inv = 1.0 / jnp.maximum(jnp.sqrt(n2), 1e-12)
    return lo * inv, hi * inv


def _tc_stage_a(B, NU, N, D, interpret=False):
    HALF = D // 2
    assert NU % 25 == 0
    BLK = NU // 25
    NBLK = NU // BLK
    f32 = jnp.float32

    def body(gsA, gs1, gs2, x0u, x1u, x2u, x3u, x0i, x1i, x2i, x3i,
             ue2n_ref, ie2n_ref, u1n_ref, i1n_ref, pu_ref, pi_ref, sup_ref):
        i = pl.program_id(0)
        q = f32(0.25)

        xu = (x0u[...] + x1u[...] + x2u[...] + x3u[...]) * q
        rl, rh = _l2h(xu[0], xu[1])
        ue2n_ref[0] = rl
        ue2n_ref[1] = rh
        xi = (x0i[...] + x1i[...] + x2i[...] + x3i[...]) * q
        sl, sh = _l2h(xi[0], xi[1])
        ie2n_ref[0] = sl
        ie2n_ref[1] = sh

        @pl.when(i == 0)
        def _():
            g1 = gs1[...]
            g2 = gs2[...]
            u1l, u1h = _l2h(g1[0, :B] * q, g1[1, :B] * q)
            i1l, i1h = _l2h(g1[0, B:] * q, g1[1, B:] * q)
            u2l, u2h = _l2h(g2[0, :B] * q, g2[1, :B] * q)
            i2l, i2h = _l2h(g2[0, B:] * q, g2[1, B:] * q)
            u1n_ref[0] = u1l
            u1n_ref[1] = u1h
            i1n_ref[0] = i1l
            i1n_ref[1] = i1h
            pu_ref[...] = jnp.sum(u1l * u2l + u1h * u2h, axis=1)[None, :]
            pi_ref[...] = jnp.sum(i1l * i2l + i1h * i2h, axis=1)[None, :]
            gA = gsA[...]
            ul, uh = gA[0, :B] * q, gA[1, :B] * q
            il, ih = gA[0, B:2 * B] * q, gA[1, B:2 * B] * q
            nl, nh = gA[0, 2 * B:] * q, gA[1, 2 * B:] * q
            sup = (jnp.sum(ul * il + uh * ih, axis=1)
                   - jnp.sum(ul * nl + uh * nh, axis=1))
            sup_ref[...] = sup[None, :]

    sds = jax.ShapeDtypeStruct
    full3 = lambda shape: pl.BlockSpec(shape, lambda i: (0, 0, 0))
    return pl.pallas_call(
        body,
        grid=(NBLK,),
        in_specs=[
            full3((NC, 3 * B, HALF)),
            full3((NC, 2 * B, HALF)),
            full3((NC, 2 * B, HALF)),
            *([pl.BlockSpec((NC, BLK, HALF), lambda i: (0, i, 0))] * 4),
            *([pl.BlockSpec((NC, BLK, HALF), lambda i: (0, i + NBLK, 0))] * 4),
        ],
        out_specs=[
            pl.BlockSpec((NC, BLK, HALF), lambda i: (0, i, 0)),
            pl.BlockSpec((NC, BLK, HALF), lambda i: (0, i, 0)),
            full3((NC, B, HALF)),
            full3((NC, B, HALF)),
            pl.BlockSpec((1, B), lambda i: (0, 0)),
            pl.BlockSpec((1, B), lambda i: (0, 0)),
            pl.BlockSpec((1, B), lambda i: (0, 0)),
        ],
        out_shape=[
            sds((NC, NU, HALF), f32), sds((NC, NU, HALF), f32),
            sds((NC, B, HALF), f32), sds((NC, B, HALF), f32),
            sds((1, B), f32), sds((1, B), f32), sds((1, B), f32),
        ],
        interpret=interpret,
    )


def _tc_matmul(B, NU, D, interpret=False):
    HALF = D // 2
    RB = 64
    assert B % RB == 0
    f32 = jnp.float32
    dn = (((1,), (1,)), ((), ()))

    def body(l1n, e2n, pos, out_ref):
        i = pl.program_id(0)
        l = l1n[...]
        r = e2n[...]
        acc = (lax.dot_general(l[0], r[0], dn, preferred_element_type=f32)
               + lax.dot_general(l[1], r[1], dn, preferred_element_type=f32))
        p = pos[i, :]
        out_ref[...] = acc - p[:, None]

    sds = jax.ShapeDtypeStruct
    return pl.pallas_call(
        body,
        grid=(B // RB,),
        in_specs=[
            pl.BlockSpec((NC, RB, HALF), lambda i: (0, i, 0)),
            pl.BlockSpec((NC, NU, HALF), lambda i: (0, 0, 0)),
            pl.BlockSpec((B // RB, RB), lambda i: (0, 0)),
        ],
        out_specs=pl.BlockSpec((RB, NU), lambda i: (i, 0)),
        out_shape=sds((B, NU), f32),
        interpret=interpret,
    )


def _run(user_table, item_table, adj_indices, adj_values, sg1_indices,
         sg1_values, sg2_indices, sg2_values, users, items, neg_items,
         interpret=False):
    NU, D = user_table.shape
    NI = item_table.shape[0]
    N = NU + NI
    E = adj_values.shape[0]
    B = users.shape[0]
    HALF = D // 2

    RPT = ((N // NS + 63) // 64) * 64
    N_PAD = RPT * NS
    ego = jnp.concatenate([user_table, item_table], axis=0)
    if N_PAD != N:
        ego = jnp.pad(ego, ((0, N_PAD - N), (0, 0)))
    x0 = jnp.stack([ego[:, :HALF], ego[:, HALF:]], axis=0)
    items_n = items + NU
    neg_n = neg_items + NU
    sel_adj = jnp.concatenate([users, items_n, neg_n])
    sel_ssl = jnp.concatenate([users, items_n])

    ts_a = _pick_sub(3 * B // NS)
    ts_s = _pick_sub(2 * B // NS)

    E_PAD = ((E + NS * CHUNK - 1) // (NS * CHUNK)) * (NS * CHUNK)

    def pe(a):
        return jnp.pad(a, (0, E_PAD - E)) if E_PAD != E else a

    gk_a = _gcn_graph_kernel(N_PAD, E_PAD, D, 3 * B, ts_a, interpret=interpret)
    gk_s = _gcn_graph_kernel(N_PAD, E_PAD, D, 2 * B, ts_s, interpret=interpret)

    _, _, _, gsA = gk_a(x0, pe(adj_indices[1]), pe(adj_indices[0]),
                        pe(adj_values), sel_adj)
    _, _, _, gs1 = gk_s(x0, pe(sg1_indices[1]), pe(sg1_indices[0]),
                        pe(sg1_values), sel_ssl)
    x1, x2, x3, gs2 = gk_s(x0, pe(sg2_indices[1]), pe(sg2_indices[0]),
                           pe(sg2_values), sel_ssl)

    ta = _tc_stage_a(B, NU, N, D, interpret=interpret)
    ue2n, ie2n, u1n, i1n, pu, pi, sup2 = ta(
        gsA, gs1, gs2, x0, x1, x2, x3, x0, x1, x2, x3)
    tm = _tc_matmul(B, NU, D, interpret=interpret)
    sslu = tm(u1n, ue2n, pu.reshape(-1, 64))
    ssli = tm(i1n, ie2n, pi.reshape(-1, 64))
    return (sup2.reshape(B), sslu, ssli)


def kernel(user_table, item_table, adj_indices, adj_values, sg1_indices,
           sg1_values, sg2_indices, sg2_values, users, items, neg_items):
    return _run(user_table, item_table, adj_indices, adj_values, sg1_indices,
                sg1_values, sg2_indices, sg2_values, users, items, neg_items)
